# baseline (speedup 1.0000x reference)
import functools

import jax
import jax.numpy as jnp
from jax import lax
from jax.experimental import pallas as pl
from jax.experimental.pallas import tpu as pltpu
from jax.experimental.pallas import tpu_sc as plsc

_N = 10000
_E = 320000
_D = 128
_G = 64

_NCORES = 2
_NTILES = 16
_CW = 128
_KI = 16
_GROUPS = 5
_CHUNKS = _KI * _GROUPS
_EPT = _CHUNKS * _CW
_ESLOTS = _NCORES * _NTILES * _EPT
_PAD_ROWS = 8
_NP = _N + _PAD_ROWS

_RPT_LO = 624
_RPT_HI = _N - 15 * _RPT_LO


def _scprop_body(h_hbm, src_hbm, dst_hbm, out_hbm,
                 agg_sh, src_va, dst_va, src_vb, dst_vb, rows_v0, rows_v1,
                 gsem0, gsem1, ssem0, ssem1, isem0, isem1, isem2, isem3):
  cid = lax.axis_index("c")
  sid = lax.axis_index("s")

  rows = (rows_v0, rows_v1)
  gsem = (gsem0, gsem1)
  ssem = (ssem0, ssem1)
  bufs = {"a": (src_va, dst_va, isem0, isem1),
          "b": (src_vb, dst_vb, isem2, isem3)}

  def stage_idx(g, which):
    sv, dv, s0, s1 = bufs[which]
    pltpu.async_copy(src_hbm.at[cid, sid, pl.ds(g * _KI, _KI)], sv, s0)
    pltpu.async_copy(dst_hbm.at[cid, sid, pl.ds(g * _KI, _KI)], dv, s1)

  def wait_idx(g, which):
    sv, dv, s0, s1 = bufs[which]
    pltpu.make_async_copy(src_hbm.at[cid, sid, pl.ds(g * _KI, _KI)],
                          sv, s0).wait()
    pltpu.make_async_copy(dst_hbm.at[cid, sid, pl.ds(g * _KI, _KI)],
                          dv, s1).wait()

  stage_idx(0, "a")
  stage_idx(1, "b")

  def zero_body(i, c):
    for k in range(8):
      rows_v0[i, pl.ds(k * 16, 16)] = jnp.zeros((16,), jnp.float32)
    return c

  lax.fori_loop(0, _CW, zero_body, 0, unroll=False)

  @pl.when(sid < 15)
  def _():
    r0 = sid * _RPT_LO
    for k in range(4):
      pltpu.sync_copy(rows_v0, agg_sh.at[pl.ds(r0 + k * _CW, _CW)])
    pltpu.sync_copy(rows_v0.at[pl.ds(0, _RPT_LO - 4 * _CW)],
                    agg_sh.at[pl.ds(r0 + 4 * _CW, _RPT_LO - 4 * _CW)])

  @pl.when(sid == 15)
  def _():
    r0 = 15 * _RPT_LO
    for k in range(5):
      pltpu.sync_copy(rows_v0, agg_sh.at[pl.ds(r0 + k * _CW, _CW)])

  @pl.when(sid == 0)
  def _():
    pltpu.sync_copy(rows_v0.at[pl.ds(0, _PAD_ROWS)],
                    agg_sh.at[pl.ds(_N, _PAD_ROWS)])

  total = _GROUPS * _KI
  which = lambda g: "a" if g % 2 == 0 else "b"
  wait_idx(0, "a")
  sva = bufs["a"][0]
  gd = [None] * total
  gd[0] = pltpu.async_copy(h_hbm.at[sva.at[0]], rows[0], gsem[0])
  gd[1] = pltpu.async_copy(h_hbm.at[sva.at[1]], rows[1], gsem[1])
  plsc.subcore_barrier()
  for t in range(total):
    g, j = divmod(t, _KI)
    dv = bufs[which(g)][1]
    b = t % 2
    gd[t].wait()
    sd = pltpu.async_copy(rows[b], agg_sh.at[dv.at[j]], ssem[b], add=True)
    sd.wait()
    if j == _KI - 1 and g + 2 <= _GROUPS - 1:
      stage_idx(g + 2, which(g))
    nt = t + 2
    if nt < total:
      ng, nj = divmod(nt, _KI)
      if nj == 0:
        wait_idx(ng, which(ng))
      sv = bufs[which(ng)][0]
      gd[nt] = pltpu.async_copy(h_hbm.at[sv.at[nj]], rows[b], gsem[b])
  plsc.subcore_barrier()

  @pl.when(sid < 15)
  def _():
    r0 = sid * _RPT_LO
    pltpu.sync_copy(agg_sh.at[pl.ds(r0, _RPT_LO)],
                    out_hbm.at[cid, pl.ds(r0, _RPT_LO)])

  @pl.when(sid == 15)
  def _():
    r0 = 15 * _RPT_LO
    pltpu.sync_copy(agg_sh.at[pl.ds(r0, _RPT_HI)],
                    out_hbm.at[cid, pl.ds(r0, _RPT_HI)])


@functools.lru_cache(maxsize=None)
def _make_scprop():
  return pl.kernel(
      _scprop_body,
      out_type=jax.ShapeDtypeStruct((_NCORES, _N, _D), jnp.float32),
      mesh=plsc.VectorSubcoreMesh(core_axis_name="c", subcore_axis_name="s"),
      scratch_types=[
          pltpu.VMEM_SHARED((_NP, _D), jnp.float32),
          pltpu.VMEM((_KI, _CW), jnp.int32),
          pltpu.VMEM((_KI, _CW), jnp.int32),
          pltpu.VMEM((_KI, _CW), jnp.int32),
          pltpu.VMEM((_KI, _CW), jnp.int32),
          pltpu.VMEM((_CW, _D), jnp.float32),
          pltpu.VMEM((_CW, _D), jnp.float32),
          pltpu.SemaphoreType.DMA,
          pltpu.SemaphoreType.DMA,
          pltpu.SemaphoreType.DMA,
          pltpu.SemaphoreType.DMA,
          pltpu.SemaphoreType.DMA,
          pltpu.SemaphoreType.DMA,
          pltpu.SemaphoreType.DMA,
          pltpu.SemaphoreType.DMA,
      ],
  )


def _dense1_body(agg_ref, x_ref, wr_ref, br_ref, wo_ref, out_ref):
  agg = agg_ref[0] + agg_ref[1]
  h = lax.dot_general(agg, wr_ref[...], (((1,), (1,)), ((), ())),
                      preferred_element_type=jnp.float32)
  h += lax.dot_general(x_ref[...], wo_ref[...], (((1,), (1,)), ((), ())),
                       preferred_element_type=jnp.float32)
  h += br_ref[...]
  out_ref[...] = jnp.maximum(h, 0.0)


def _dense2_body(agg_ref, h1_ref, batch_ref, wr_ref, br_ref, wo_ref,
                 wp1_ref, bp1_ref, wp2_ref, bp2_ref, out_ref,
                 pooled_acc, cnt_acc):
  i = pl.program_id(0)
  nsteps = pl.num_programs(0)

  @pl.when(i == 0)
  def _():
    pooled_acc[...] = jnp.zeros_like(pooled_acc)
    cnt_acc[...] = jnp.zeros_like(cnt_acc)

  agg = agg_ref[0] + agg_ref[1]
  h2 = lax.dot_general(agg, wr_ref[...], (((1,), (1,)), ((), ())),
                       preferred_element_type=jnp.float32)
  h2 += lax.dot_general(h1_ref[...], wo_ref[...], (((1,), (1,)), ((), ())),
                        preferred_element_type=jnp.float32)
  h2 += br_ref[...]
  h2 = jnp.maximum(h2, 0.0)

  bn = h2.shape[0]
  iota_g = lax.broadcasted_iota(jnp.int32, (_G, bn), 0)
  mask = (batch_ref[0] == iota_g).astype(jnp.float32)
  pooled_acc[...] += lax.dot_general(mask, h2, (((1,), (0,)), ((), ())),
                                     preferred_element_type=jnp.float32)
  cnt_acc[...] += jnp.sum(mask, axis=1, keepdims=True)

  @pl.when(i == nsteps - 1)
  def _():
    pooled = pooled_acc[...] / jnp.maximum(cnt_acc[...], 1.0)
    o = lax.dot_general(pooled, wp1_ref[...], (((1,), (1,)), ((), ())),
                        preferred_element_type=jnp.float32)
    o += bp1_ref[...]
    o = lax.dot_general(o, wp2_ref[...], (((1,), (1,)), ((), ())),
                        preferred_element_type=jnp.float32)
    o += bp2_ref[...]
    out_ref[...] = o


_BN = 2000
_GRID = _N // _BN


def _dense1(agg, x, wr, br, wo):
  return pl.pallas_call(
      _dense1_body,
      grid=(_GRID,),
      in_specs=[
          pl.BlockSpec((_NCORES, _BN, _D), lambda i: (0, i, 0)),
          pl.BlockSpec((_BN, _D), lambda i: (i, 0)),
          pl.BlockSpec((_D, _D), lambda i: (0, 0)),
          pl.BlockSpec((1, _D), lambda i: (0, 0)),
          pl.BlockSpec((_D, _D), lambda i: (0, 0)),
      ],
      out_specs=pl.BlockSpec((_BN, _D), lambda i: (i, 0)),
      out_shape=jax.ShapeDtypeStruct((_N, _D), jnp.float32),
  )(agg, x, wr, br, wo)


def _dense2(agg, h1, batch3d, wr, br, wo, wp1, bp1, wp2, bp2):
  return pl.pallas_call(
      _dense2_body,
      grid=(_GRID,),
      in_specs=[
          pl.BlockSpec((_NCORES, _BN, _D), lambda i: (0, i, 0)),
          pl.BlockSpec((_BN, _D), lambda i: (i, 0)),
          pl.BlockSpec((1, 1, _BN), lambda i: (i, 0, 0)),
          pl.BlockSpec((_D, _D), lambda i: (0, 0)),
          pl.BlockSpec((1, _D), lambda i: (0, 0)),
          pl.BlockSpec((_D, _D), lambda i: (0, 0)),
          pl.BlockSpec((_D, _D), lambda i: (0, 0)),
          pl.BlockSpec((1, _D), lambda i: (0, 0)),
          pl.BlockSpec((_D, _D), lambda i: (0, 0)),
          pl.BlockSpec((1, _D), lambda i: (0, 0)),
      ],
      out_specs=pl.BlockSpec((_G, _D), lambda i: (0, 0)),
      out_shape=jax.ShapeDtypeStruct((_G, _D), jnp.float32),
      scratch_shapes=[
          pltpu.VMEM((_G, _D), jnp.float32),
          pltpu.VMEM((_G, 1), jnp.float32),
      ],
  )(agg, h1, batch3d, wr, br, wo, wp1, bp1, wp2, bp2)


@jax.jit
def kernel(x, edge_index, batch, W_rel0, b_rel0, W_root0, W_rel1, b_rel1,
           W_root1, Wp1, bp1, Wp2, bp2):
  nw = _NCORES * _NTILES
  ept_real = _E // nw
  npad_t = _EPT - ept_real
  pad_src = (jnp.arange(nw * npad_t, dtype=jnp.int32) % 4096).reshape(
      nw, npad_t)
  pad_dst = _N + (jnp.arange(nw * npad_t, dtype=jnp.int32)
                  % _PAD_ROWS).reshape(nw, npad_t)
  srcp = jnp.concatenate(
      [edge_index[0].reshape(nw, ept_real), pad_src], axis=1).reshape(
          _NCORES, _NTILES, _CHUNKS, _CW)
  dstp = jnp.concatenate(
      [edge_index[1].reshape(nw, ept_real), pad_dst], axis=1).reshape(
          _NCORES, _NTILES, _CHUNKS, _CW)
  batch3d = batch.reshape(_GRID, 1, _BN)
  br0 = b_rel0.reshape(1, _D)
  br1 = b_rel1.reshape(1, _D)
  bp1r = bp1.reshape(1, _D)
  bp2r = bp2.reshape(1, _D)

  scprop = _make_scprop()
  agg0 = scprop(x, srcp, dstp)
  h1 = _dense1(agg0, x, W_rel0, br0, W_root0)
  agg1 = scprop(h1, srcp, dstp)
  out = _dense2(agg1, h1, batch3d, W_rel1, br1, W_root1,
                Wp1, bp1r, Wp2, bp2r)
  return out


# speedup vs baseline: 1.0059x; 1.0059x over previous
import functools

import jax
import jax.numpy as jnp
from jax import lax
from jax.experimental import pallas as pl
from jax.experimental.pallas import tpu as pltpu
from jax.experimental.pallas import tpu_sc as plsc

_N = 10000
_E = 320000
_D = 128
_G = 64

_NCORES = 2
_NTILES = 16
_CW = 128
_KI = 16
_GROUPS = 5
_CHUNKS = _KI * _GROUPS
_EPT = _CHUNKS * _CW
_ESLOTS = _NCORES * _NTILES * _EPT
_PAD_ROWS = 8
_NP = _N + _PAD_ROWS

_RPT_LO = 624
_RPT_HI = _N - 15 * _RPT_LO


def _scprop_body(h_hbm, src_hbm, dst_hbm, out_hbm,
                 agg_sh, src_va, dst_va, src_vb, dst_vb, rows_v0, rows_v1,
                 gsem0, gsem1, ssem0, ssem1, isem0, isem1, isem2, isem3):
  cid = lax.axis_index("c")
  sid = lax.axis_index("s")

  rows = (rows_v0, rows_v1)
  gsem = (gsem0, gsem1)
  ssem = (ssem0, ssem1)
  bufs = {"a": (src_va, dst_va, isem0, isem1),
          "b": (src_vb, dst_vb, isem2, isem3)}

  def stage_idx(g, which):
    sv, dv, s0, s1 = bufs[which]
    pltpu.async_copy(src_hbm.at[cid, sid, pl.ds(g * _KI, _KI)], sv, s0)
    pltpu.async_copy(dst_hbm.at[cid, sid, pl.ds(g * _KI, _KI)], dv, s1)

  def wait_idx(g, which):
    sv, dv, s0, s1 = bufs[which]
    pltpu.make_async_copy(src_hbm.at[cid, sid, pl.ds(g * _KI, _KI)],
                          sv, s0).wait()
    pltpu.make_async_copy(dst_hbm.at[cid, sid, pl.ds(g * _KI, _KI)],
                          dv, s1).wait()

  stage_idx(0, "a")
  stage_idx(1, "b")

  def zero_body(i, c):
    for k in range(8):
      rows_v0[i, pl.ds(k * 16, 16)] = jnp.zeros((16,), jnp.float32)
    return c

  lax.fori_loop(0, _CW, zero_body, 0, unroll=False)

  @pl.when(sid < 15)
  def _():
    r0 = sid * _RPT_LO
    for k in range(4):
      pltpu.sync_copy(rows_v0, agg_sh.at[pl.ds(r0 + k * _CW, _CW)])
    pltpu.sync_copy(rows_v0.at[pl.ds(0, _RPT_LO - 4 * _CW)],
                    agg_sh.at[pl.ds(r0 + 4 * _CW, _RPT_LO - 4 * _CW)])

  @pl.when(sid == 15)
  def _():
    r0 = 15 * _RPT_LO
    for k in range(5):
      pltpu.sync_copy(rows_v0, agg_sh.at[pl.ds(r0 + k * _CW, _CW)])

  @pl.when(sid == 0)
  def _():
    pltpu.sync_copy(rows_v0.at[pl.ds(0, _PAD_ROWS)],
                    agg_sh.at[pl.ds(_N, _PAD_ROWS)])

  total = _GROUPS * _KI
  which = lambda g: "a" if g % 2 == 0 else "b"
  wait_idx(0, "a")
  sva = bufs["a"][0]
  gd = [None] * total
  gd[0] = pltpu.async_copy(h_hbm.at[sva.at[0]], rows[0], gsem[0])
  gd[1] = pltpu.async_copy(h_hbm.at[sva.at[1]], rows[1], gsem[1])
  plsc.subcore_barrier()
  for t in range(total):
    g, j = divmod(t, _KI)
    dv = bufs[which(g)][1]
    b = t % 2
    gd[t].wait()
    sd = pltpu.async_copy(rows[b], agg_sh.at[dv.at[j]], ssem[b], add=True)
    sd.wait()
    if j == _KI - 1 and g + 2 <= _GROUPS - 1:
      stage_idx(g + 2, which(g))
    nt = t + 2
    if nt < total:
      ng, nj = divmod(nt, _KI)
      if nj == 0:
        wait_idx(ng, which(ng))
      sv = bufs[which(ng)][0]
      gd[nt] = pltpu.async_copy(h_hbm.at[sv.at[nj]], rows[b], gsem[b])
  plsc.subcore_barrier()

  @pl.when(sid < 15)
  def _():
    r0 = sid * _RPT_LO
    pltpu.sync_copy(agg_sh.at[pl.ds(r0, _RPT_LO)],
                    out_hbm.at[cid, pl.ds(r0, _RPT_LO)])

  @pl.when(sid == 15)
  def _():
    r0 = 15 * _RPT_LO
    pltpu.sync_copy(agg_sh.at[pl.ds(r0, _RPT_HI)],
                    out_hbm.at[cid, pl.ds(r0, _RPT_HI)])


@functools.lru_cache(maxsize=None)
def _make_scprop():
  return pl.kernel(
      _scprop_body,
      out_type=jax.ShapeDtypeStruct((_NCORES, _N, _D), jnp.float32),
      mesh=plsc.VectorSubcoreMesh(core_axis_name="c", subcore_axis_name="s"),
      scratch_types=[
          pltpu.VMEM_SHARED((_NP, _D), jnp.float32),
          pltpu.VMEM((_KI, _CW), jnp.int32),
          pltpu.VMEM((_KI, _CW), jnp.int32),
          pltpu.VMEM((_KI, _CW), jnp.int32),
          pltpu.VMEM((_KI, _CW), jnp.int32),
          pltpu.VMEM((_CW, _D), jnp.float32),
          pltpu.VMEM((_CW, _D), jnp.float32),
          pltpu.SemaphoreType.DMA,
          pltpu.SemaphoreType.DMA,
          pltpu.SemaphoreType.DMA,
          pltpu.SemaphoreType.DMA,
          pltpu.SemaphoreType.DMA,
          pltpu.SemaphoreType.DMA,
          pltpu.SemaphoreType.DMA,
          pltpu.SemaphoreType.DMA,
      ],
  )


def _dense1_body(agg_ref, x_ref, wr_ref, br_ref, wo_ref, out_ref):
  agg = agg_ref[0] + agg_ref[1]
  h = lax.dot_general(agg, wr_ref[...], (((1,), (1,)), ((), ())),
                      preferred_element_type=jnp.float32)
  h += lax.dot_general(x_ref[...], wo_ref[...], (((1,), (1,)), ((), ())),
                       preferred_element_type=jnp.float32)
  h += br_ref[...]
  out_ref[...] = jnp.maximum(h, 0.0)


def _dense2_body(agg_ref, h1_ref, batch_ref, wr_ref, br_ref, wo_ref,
                 wp1_ref, bp1_ref, wp2_ref, bp2_ref, out_ref,
                 pooled_acc, cnt_acc):
  i = pl.program_id(0)
  nsteps = pl.num_programs(0)

  @pl.when(i == 0)
  def _():
    pooled_acc[...] = jnp.zeros_like(pooled_acc)
    cnt_acc[...] = jnp.zeros_like(cnt_acc)

  agg = agg_ref[0] + agg_ref[1]
  h2 = lax.dot_general(agg, wr_ref[...], (((1,), (1,)), ((), ())),
                       preferred_element_type=jnp.float32)
  h2 += lax.dot_general(h1_ref[...], wo_ref[...], (((1,), (1,)), ((), ())),
                        preferred_element_type=jnp.float32)
  h2 += br_ref[...]
  h2 = jnp.maximum(h2, 0.0)

  bn = h2.shape[0]
  iota_g = lax.broadcasted_iota(jnp.int32, (_G, bn), 0)
  mask = (batch_ref[0] == iota_g).astype(jnp.float32)
  pooled_acc[...] += lax.dot_general(mask, h2, (((1,), (0,)), ((), ())),
                                     preferred_element_type=jnp.float32)
  cnt_acc[...] += jnp.sum(mask, axis=1, keepdims=True)

  @pl.when(i == nsteps - 1)
  def _():
    pooled = pooled_acc[...] / jnp.maximum(cnt_acc[...], 1.0)
    o = lax.dot_general(pooled, wp1_ref[...], (((1,), (1,)), ((), ())),
                        preferred_element_type=jnp.float32)
    o += bp1_ref[...]
    o = lax.dot_general(o, wp2_ref[...], (((1,), (1,)), ((), ())),
                        preferred_element_type=jnp.float32)
    o += bp2_ref[...]
    out_ref[...] = o


_BN = 2000
_GRID = _N // _BN


def _dense1(agg, x, wr, br, wo):
  return pl.pallas_call(
      _dense1_body,
      grid=(_GRID,),
      in_specs=[
          pl.BlockSpec((_NCORES, _BN, _D), lambda i: (0, i, 0)),
          pl.BlockSpec((_BN, _D), lambda i: (i, 0)),
          pl.BlockSpec((_D, _D), lambda i: (0, 0)),
          pl.BlockSpec((1, _D), lambda i: (0, 0)),
          pl.BlockSpec((_D, _D), lambda i: (0, 0)),
      ],
      out_specs=pl.BlockSpec((_BN, _D), lambda i: (i, 0)),
      out_shape=jax.ShapeDtypeStruct((_N, _D), jnp.float32),
  )(agg, x, wr, br, wo)


def _dense2(agg, h1, batch3d, wr, br, wo, wp1, bp1, wp2, bp2):
  return pl.pallas_call(
      _dense2_body,
      grid=(_GRID,),
      in_specs=[
          pl.BlockSpec((_NCORES, _BN, _D), lambda i: (0, i, 0)),
          pl.BlockSpec((_BN, _D), lambda i: (i, 0)),
          pl.BlockSpec((1, 1, _BN), lambda i: (i, 0, 0)),
          pl.BlockSpec((_D, _D), lambda i: (0, 0)),
          pl.BlockSpec((1, _D), lambda i: (0, 0)),
          pl.BlockSpec((_D, _D), lambda i: (0, 0)),
          pl.BlockSpec((_D, _D), lambda i: (0, 0)),
          pl.BlockSpec((1, _D), lambda i: (0, 0)),
          pl.BlockSpec((_D, _D), lambda i: (0, 0)),
          pl.BlockSpec((1, _D), lambda i: (0, 0)),
      ],
      out_specs=pl.BlockSpec((_G, _D), lambda i: (0, 0)),
      out_shape=jax.ShapeDtypeStruct((_G, _D), jnp.float32),
      scratch_shapes=[
          pltpu.VMEM((_G, _D), jnp.float32),
          pltpu.VMEM((_G, 1), jnp.float32),
      ],
  )(agg, h1, batch3d, wr, br, wo, wp1, bp1, wp2, bp2)


@jax.jit
def kernel(x, edge_index, batch, W_rel0, b_rel0, W_root0, W_rel1, b_rel1,
           W_root1, Wp1, bp1, Wp2, bp2):
  npad = _ESLOTS - _E
  pad_src = jnp.arange(npad, dtype=jnp.int32) % 4096
  pad_dst = _N + (jnp.arange(npad, dtype=jnp.int32) % _PAD_ROWS)
  srcp = jnp.concatenate([edge_index[0], pad_src]).reshape(
      _NCORES, _NTILES, _CHUNKS, _CW)
  dstp = jnp.concatenate([edge_index[1], pad_dst]).reshape(
      _NCORES, _NTILES, _CHUNKS, _CW)
  batch3d = batch.reshape(_GRID, 1, _BN)
  br0 = b_rel0.reshape(1, _D)
  br1 = b_rel1.reshape(1, _D)
  bp1r = bp1.reshape(1, _D)
  bp2r = bp2.reshape(1, _D)

  scprop = _make_scprop()
  agg0 = scprop(x, srcp, dstp)
  h1 = _dense1(agg0, x, W_rel0, br0, W_root0)
  agg1 = scprop(h1, srcp, dstp)
  out = _dense2(agg1, h1, batch3d, W_rel1, br1, W_root1,
                Wp1, bp1r, Wp2, bp2r)
  return out
